# baseline (speedup 1.0000x reference)
import functools

import jax
import jax.numpy as jnp
from jax import lax
from jax.experimental import pallas as pl
from jax.experimental.pallas import tpu as pltpu
from jax.experimental.pallas import tpu_sc as plsc

_NC, _NS = 2, 16
_NW = _NC * _NS
_LANES = 16


def _tc_pack(pe3, ms0, ms1, h):
    msp = ((ms1 + 7) // 8) * 8
    sb = 20
    hh = h // 2

    def body(pe_ref, o_ref):
        for k in range(sb):
            v = pe_ref[k].astype(jnp.bfloat16)
            lo = jax.lax.bitcast_convert_type(v[:, :hh], jnp.uint16).astype(
                jnp.int32
            )
            hi = jax.lax.bitcast_convert_type(v[:, hh:], jnp.uint16).astype(
                jnp.int32
            )
            o_ref[k * msp : k * msp + ms1, :] = lo | (hi << 16)

    return pl.pallas_call(
        body,
        grid=(ms0 // sb,),
        in_specs=[pl.BlockSpec((sb, ms1, h), lambda i: (i, 0, 0))],
        out_specs=pl.BlockSpec((sb * msp, hh), lambda i: (i, 0)),
        out_shape=jax.ShapeDtypeStruct((ms0 * msp, hh), jnp.int32),
    )(pe3)


def _sc_gather(c0, c1, pe_flat, ms0, ms1, msp, hh, n, row0, nk):
    b_per_w = nk // _NW
    ch = 64
    nch = b_per_w // ch
    mesh = plsc.VectorSubcoreMesh(core_axis_name="c", subcore_axis_name="s")

    @functools.partial(
        pl.kernel,
        mesh=mesh,
        out_type=jax.ShapeDtypeStruct((nk, hh), jnp.int32),
        scratch_types=[
            pltpu.VMEM((b_per_w,), jnp.float32),
            pltpu.VMEM((b_per_w,), jnp.float32),
            pltpu.VMEM((b_per_w,), jnp.int32),
            pltpu.VMEM((ch, hh), jnp.int32),
            pltpu.VMEM((ch, hh), jnp.int32),
            pltpu.SemaphoreType.DMA,
            pltpu.SemaphoreType.DMA,
        ],
    )
    def k(c0_hbm, c1_hbm, pe_hbm, enc_hbm, c0_v, c1_v, idx_v, rows0, rows1,
          sem0, sem1):
        wid = lax.axis_index("s") * _NC + lax.axis_index("c")
        base = wid * b_per_w
        pltpu.sync_copy(c0_hbm.at[pl.ds(row0 + base, b_per_w)], c0_v)
        pltpu.sync_copy(c1_hbm.at[pl.ds(row0 + base, b_per_w)], c1_v)

        @pl.loop(0, b_per_w, step=_LANES)
        def _(i):
            a = (c0_v[pl.ds(i, _LANES)] * float(ms0 - 1)).astype(jnp.int32)
            b = (c1_v[pl.ds(i, _LANES)] * float(ms1 - 1)).astype(jnp.int32)
            idx_v[pl.ds(i, _LANES)] = a * msp + b

        bufs = (rows0, rows1)
        sems = (sem0, sem1)

        def start(c):
            pltpu.async_copy(
                pe_hbm.at[idx_v.at[pl.ds(c * ch, ch)]],
                bufs[c % 2],
                sems[c % 2],
            )

        start(0)
        for c in range(nch):
            if c + 1 < nch:
                start(c + 1)
            pltpu.make_async_copy(
                pe_hbm.at[idx_v.at[pl.ds(c * ch, ch)]],
                bufs[c % 2],
                sems[c % 2],
            ).wait()
            pltpu.sync_copy(
                bufs[c % 2], enc_hbm.at[pl.ds(base + c * ch, ch)]
            )

    return k(c0, c1, pe_flat)


def _tc_combine(x2d, enc_k, te_row, out_init, n, d, h, row0, nk):
    rb = 1024
    hh = h // 2
    off = row0 // rb

    def body(*refs):
        x_ref, enc_ref, te_ref, o_ref = refs[-4:]
        u = enc_ref[...]
        lo = jax.lax.bitcast_convert_type(u << 16, jnp.float32)
        hi = jax.lax.bitcast_convert_type(u & jnp.int32(-65536), jnp.float32)
        o_ref[:, :hh] = x_ref[:, :hh] + lo
        o_ref[:, hh : 2 * hh] = x_ref[:, hh : 2 * hh] + hi
        o_ref[:, 2 * hh :] = x_ref[:, 2 * hh :] + te_ref[...]

    in_specs = [
        pl.BlockSpec((rb, d), lambda i, off=off: (i + off, 0)),
        pl.BlockSpec((rb, hh), lambda i: (i, 0)),
        pl.BlockSpec((1, h), lambda i: (0, 0)),
    ]
    args = [x2d, enc_k, te_row]
    aliases = {}
    if out_init is not None:
        in_specs = [pl.BlockSpec(memory_space=pl.ANY)] + in_specs
        args = [out_init] + args
        aliases = {0: 0}

    return pl.pallas_call(
        body,
        grid=(nk // rb,),
        in_specs=in_specs,
        out_specs=pl.BlockSpec((rb, d), lambda i, off=off: (i + off, 0)),
        out_shape=jax.ShapeDtypeStruct((n, d), jnp.float32),
        input_output_aliases=aliases,
    )(*args)


def kernel(x, spatial_coords, spatial_pe, temporal_pe, temporal_idx):
    B, S, D = x.shape
    H = D // 2
    HH = H // 2
    N = B * S
    MS0, MS1 = spatial_pe.shape[0], spatial_pe.shape[1]
    MSP = ((MS1 + 7) // 8) * 8
    MT = temporal_pe.shape[0]

    c0 = spatial_coords[..., 0].reshape(N)
    c1 = spatial_coords[..., 1].reshape(N)
    x2d = x.reshape(N, D)

    pe_packed = _tc_pack(spatial_pe, MS0, MS1, H)
    te_row = jax.lax.dynamic_slice_in_dim(temporal_pe, temporal_idx % MT, 1)

    sizes = [N // 8, N // 4, N // 4, N * 3 // 8]
    row0s = [0, N // 8, N * 3 // 8, N * 5 // 8]
    encs = [
        _sc_gather(c0, c1, pe_packed, MS0, MS1, MSP, HH, N, r0, nk)
        for r0, nk in zip(row0s, sizes)
    ]
    out = None
    for r0, nk, enc in zip(row0s, sizes, encs):
        out = _tc_combine(x2d, enc, te_row, out, N, D, H, r0, nk)
    return out.reshape(B, S, D)


# speedup vs baseline: 1.0243x; 1.0243x over previous
import functools

import jax
import jax.numpy as jnp
from jax import lax
from jax.experimental import pallas as pl
from jax.experimental.pallas import tpu as pltpu
from jax.experimental.pallas import tpu_sc as plsc

_NC, _NS = 2, 16
_NW = _NC * _NS
_LANES = 16


def _tc_pack(pe3, ms0, ms1, h):
    msp = ((ms1 + 7) // 8) * 8
    sb = 20
    hh = h // 2

    def body(pe_ref, o_ref):
        for k in range(sb):
            v = pe_ref[k].astype(jnp.bfloat16)
            lo = jax.lax.bitcast_convert_type(v[:, :hh], jnp.uint16).astype(
                jnp.int32
            )
            hi = jax.lax.bitcast_convert_type(v[:, hh:], jnp.uint16).astype(
                jnp.int32
            )
            o_ref[k * msp : k * msp + ms1, :] = lo | (hi << 16)

    return pl.pallas_call(
        body,
        grid=(ms0 // sb,),
        in_specs=[pl.BlockSpec((sb, ms1, h), lambda i: (i, 0, 0))],
        out_specs=pl.BlockSpec((sb * msp, hh), lambda i: (i, 0)),
        out_shape=jax.ShapeDtypeStruct((ms0 * msp, hh), jnp.int32),
    )(pe3)


def _sc_gather(c0, c1, pe_flat, ms0, ms1, msp, hh, n, row0, nk):
    b_per_w = nk // _NW
    ch = 64
    nch = b_per_w // ch
    mesh = plsc.VectorSubcoreMesh(core_axis_name="c", subcore_axis_name="s")

    @functools.partial(
        pl.kernel,
        mesh=mesh,
        out_type=jax.ShapeDtypeStruct((nk, hh), jnp.int32),
        scratch_types=[
            pltpu.VMEM((b_per_w,), jnp.float32),
            pltpu.VMEM((b_per_w,), jnp.float32),
            pltpu.VMEM((b_per_w,), jnp.int32),
            pltpu.VMEM((ch, hh), jnp.int32),
            pltpu.VMEM((ch, hh), jnp.int32),
            pltpu.SemaphoreType.DMA,
            pltpu.SemaphoreType.DMA,
        ],
    )
    def k(c0_hbm, c1_hbm, pe_hbm, enc_hbm, c0_v, c1_v, idx_v, rows0, rows1,
          sem0, sem1):
        wid = lax.axis_index("s") * _NC + lax.axis_index("c")
        base = wid * b_per_w
        pltpu.sync_copy(c0_hbm.at[pl.ds(row0 + base, b_per_w)], c0_v)
        pltpu.sync_copy(c1_hbm.at[pl.ds(row0 + base, b_per_w)], c1_v)

        @pl.loop(0, b_per_w, step=_LANES)
        def _(i):
            a = (c0_v[pl.ds(i, _LANES)] * float(ms0 - 1)).astype(jnp.int32)
            b = (c1_v[pl.ds(i, _LANES)] * float(ms1 - 1)).astype(jnp.int32)
            idx_v[pl.ds(i, _LANES)] = a * msp + b

        bufs = (rows0, rows1)
        sems = (sem0, sem1)

        def start(c):
            pltpu.async_copy(
                pe_hbm.at[idx_v.at[pl.ds(c * ch, ch)]],
                bufs[c % 2],
                sems[c % 2],
            )

        start(0)
        for c in range(nch):
            if c + 1 < nch:
                start(c + 1)
            pltpu.make_async_copy(
                pe_hbm.at[idx_v.at[pl.ds(c * ch, ch)]],
                bufs[c % 2],
                sems[c % 2],
            ).wait()
            pltpu.sync_copy(
                bufs[c % 2], enc_hbm.at[pl.ds(base + c * ch, ch)]
            )

    return k(c0, c1, pe_flat)


def _tc_combine(x2d, enc_k, te_row, out_init, n, d, h, row0, nk):
    rb = 1024
    hh = h // 2
    off = row0 // rb

    def body(*refs):
        x_ref, enc_ref, te_ref, o_ref = refs[-4:]
        u = enc_ref[...]
        lo = jax.lax.bitcast_convert_type(u << 16, jnp.float32)
        hi = jax.lax.bitcast_convert_type(u & jnp.int32(-65536), jnp.float32)
        o_ref[:, :hh] = x_ref[:, :hh] + lo
        o_ref[:, hh : 2 * hh] = x_ref[:, hh : 2 * hh] + hi
        o_ref[:, 2 * hh :] = x_ref[:, 2 * hh :] + te_ref[...]

    in_specs = [
        pl.BlockSpec((rb, d), lambda i, off=off: (i + off, 0)),
        pl.BlockSpec((rb, hh), lambda i: (i, 0)),
        pl.BlockSpec((1, h), lambda i: (0, 0)),
    ]
    args = [x2d, enc_k, te_row]
    aliases = {}
    if out_init is not None:
        in_specs = [pl.BlockSpec(memory_space=pl.ANY)] + in_specs
        args = [out_init] + args
        aliases = {0: 0}

    return pl.pallas_call(
        body,
        grid=(nk // rb,),
        in_specs=in_specs,
        out_specs=pl.BlockSpec((rb, d), lambda i, off=off: (i + off, 0)),
        out_shape=jax.ShapeDtypeStruct((n, d), jnp.float32),
        input_output_aliases=aliases,
    )(*args)


def kernel(x, spatial_coords, spatial_pe, temporal_pe, temporal_idx):
    B, S, D = x.shape
    H = D // 2
    HH = H // 2
    N = B * S
    MS0, MS1 = spatial_pe.shape[0], spatial_pe.shape[1]
    MSP = ((MS1 + 7) // 8) * 8
    MT = temporal_pe.shape[0]

    c0 = spatial_coords[..., 0].reshape(N)
    c1 = spatial_coords[..., 1].reshape(N)
    x2d = x.reshape(N, D)

    pe_packed = _tc_pack(spatial_pe, MS0, MS1, H)
    te_row = jax.lax.dynamic_slice_in_dim(temporal_pe, temporal_idx % MT, 1)

    sizes = [N // 4, N // 4, N // 2]
    row0s = [0, N // 4, N // 2]
    encs = [
        _sc_gather(c0, c1, pe_packed, MS0, MS1, MSP, HH, N, r0, nk)
        for r0, nk in zip(row0s, sizes)
    ]
    out = None
    for r0, nk, enc in zip(row0s, sizes, encs):
        out = _tc_combine(x2d, enc, te_row, out, N, D, H, r0, nk)
    return out.reshape(B, S, D)
